# baseline (speedup 1.0000x reference)
import functools

import jax
import jax.numpy as jnp
import numpy as np
from jax import lax
from jax.experimental import pallas as pl
from jax.experimental.pallas import tpu as pltpu
from jax.experimental.pallas import tpu_sc as plsc


_B, _S, _D = 2, 4096, 4096
_BS = 4096
_DK = 1024
_K = 512
_NV = _S // 16


def _matvec_body(x_ref, w_ref, o_ref):
    def _dot(ww, xx):
        return jax.lax.dot_general(
            ww, xx,
            dimension_numbers=(((1,), (1,)), ((), ())),
            preferred_element_type=jnp.float32,
        )

    def _chunk(c):
        lo, hi = 256 * c, 256 * c + 128
        return (_dot(w_ref[:, lo:lo + 128], x_ref[0, :, lo:lo + 128])
                + _dot(w_ref[:, hi:hi + 128], x_ref[0, :, hi:hi + 128]))

    k = pl.program_id(2)

    @pl.when(k == 0)
    def _init():
        a = _chunk(0)
        for c in range(1, _DK // 256):
            a = a + _chunk(c)
        o_ref[0] = a

    @pl.when(k != 0)
    def _acc():
        a = o_ref[0]
        for c in range(_DK // 256):
            a = a + _chunk(c)
        o_ref[0] = a


def _router_logits(hidden_states, W_router):
    wt = W_router.T
    nb = hidden_states.shape[0]
    out = pl.pallas_call(
        _matvec_body,
        grid=(nb, _S // _BS, _D // _DK),
        in_specs=[
            pl.BlockSpec((1, _BS, _DK), lambda b, s, k: (b, s, k)),
            pl.BlockSpec((1, _DK), lambda b, s, k: (0, k)),
        ],
        out_specs=pl.BlockSpec((1, 1, _BS), lambda b, s, k: (b, 0, s)),
        out_shape=jax.ShapeDtypeStruct((nb, 1, _S), jnp.float32),
    )(hidden_states, wt)
    return out[:, 0, :]


def _radix_pass(src_k, src_i, dst_k, dst_i, hist, shift, nbuckets, nveg):
    mask = jnp.int32(nbuckets - 1)
    ones = jnp.ones((16,), jnp.int32)

    for j in range(nbuckets // 16):
        hist[pl.ds(j * 16, 16)] = jnp.zeros((16,), jnp.int32)

    def _hist(i, carry):
        k = src_k[pl.ds(i * 16, 16)]
        d = lax.bitwise_and(lax.shift_right_logical(k, shift), mask)
        plsc.addupdate_scatter(hist, [d], ones)
        return carry

    lax.fori_loop(0, nveg, _hist, 0)

    def _scan(j, carry):
        h = hist[pl.ds(j * 16, 16)]
        inc = plsc.cumsum(h)
        hist[pl.ds(j * 16, 16)] = inc - h + carry
        return carry + jnp.sum(h)

    lax.fori_loop(0, nbuckets // 16, _scan, jnp.int32(0))

    def _scatter(i, carry):
        k = src_k[pl.ds(i * 16, 16)]
        v = src_i[pl.ds(i * 16, 16)]
        d = lax.bitwise_and(lax.shift_right_logical(k, shift), mask)
        cnt, _ = plsc.scan_count(d)
        base = plsc.load_gather(hist, [d])
        p = base + cnt - 1
        plsc.store_scatter(dst_k, [p], k)
        plsc.store_scatter(dst_i, [p], v)
        plsc.addupdate_scatter(hist, [d], ones)
        return carry

    lax.fori_loop(0, nveg, _scatter, 0)


_MIN32 = np.int32(-(2 ** 31))


def _routing_body(logits_hbm, topk_hbm, sorted_hbm, w_hbm,
                  vals, ka, sk, si, sv, rk, ri, hist):
    c = lax.axis_index("c")
    s = lax.axis_index("s")
    nb = logits_hbm.shape[0]

    @pl.when((s == 0) & (c < nb))
    def _route():
        row = c
        pltpu.sync_copy(logits_hbm.at[row], vals)

        def _keys(i, carry):
            bits = plsc.bitcast(vals[pl.ds(i * 16, 16)], jnp.int32)
            sgn = lax.shift_right_arithmetic(bits, 31)
            srt = lax.bitwise_xor(bits, lax.bitwise_or(sgn, _MIN32))
            ka[pl.ds(i * 16, 16)] = lax.bitwise_not(srt)
            return carry

        lax.fori_loop(0, _NV, _keys, 0)

        ones = jnp.ones((16,), jnp.int32)

        def _level(shift, maskhi, pref, need):
            for j in range(16):
                hist[pl.ds(j * 16, 16)] = jnp.zeros((16,), jnp.int32)

            def _h(i, carry):
                k = ka[pl.ds(i * 16, 16)]
                match = lax.bitwise_and(k, maskhi) == pref
                d = lax.bitwise_and(lax.shift_right_logical(k, shift),
                                    jnp.int32(255))
                plsc.addupdate_scatter(hist, [d], ones, mask=match)
                return carry

            lax.fori_loop(0, _NV, _h, 0)

            def _scan(j, carry):
                cum, fb, fe = carry
                h = hist[pl.ds(j * 16, 16)]
                inc = plsc.cumsum(h)
                excl = inc - h + cum
                hit = (excl < need) & (excl + h >= need)
                lane = lax.iota(jnp.int32, 16) + j * 16
                fb = jnp.maximum(fb, jnp.max(jnp.where(hit, lane, -1)))
                fe = jnp.maximum(fe, jnp.max(jnp.where(hit, excl, -1)))
                return cum + jnp.sum(h), fb, fe

            _, fb, fe = lax.fori_loop(
                0, 16, _scan, (jnp.int32(0), jnp.int32(-1), jnp.int32(-1)))
            return pref | lax.shift_left(fb, shift), need - fe

        pref, need = jnp.int32(0), jnp.int32(_K)
        pref, need = _level(24, jnp.int32(0), pref, need)
        pref, need = _level(16, jnp.int32(-(2 ** 24)), pref, need)
        pref, need = _level(8, jnp.int32(-(2 ** 16)), pref, need)
        pref, need = _level(0, jnp.int32(-(2 ** 8)), pref, need)

        tf = lax.bitwise_xor(pref, _MIN32)
        needv = jnp.full((16,), 1, jnp.int32) * need

        def _compact(i, carry):
            base, eqb = carry
            k = ka[pl.ds(i * 16, 16)]
            v = vals[pl.ds(i * 16, 16)]
            idx = lax.iota(jnp.int32, 16) + i * 16
            m_lt = lax.bitwise_xor(k, _MIN32) < tf
            m_eq = k == pref
            ceq = plsc.cumsum(m_eq.astype(jnp.int32))
            m = m_lt | (m_eq & ((eqb + ceq) <= needv))
            p = base + plsc.cumsum(m.astype(jnp.int32)) - 1
            plsc.store_scatter(sk, [p], k, mask=m)
            plsc.store_scatter(si, [p], idx, mask=m)
            plsc.store_scatter(sv, [p], v, mask=m)
            return (base + plsc.all_reduce_population_count(m),
                    eqb + plsc.all_reduce_population_count(m_eq))

        lax.fori_loop(0, _NV, _compact,
                      (jnp.zeros((16,), jnp.int32), jnp.zeros((16,), jnp.int32)))

        pltpu.sync_copy(si, sorted_hbm.at[row])

        def _gmax(i, m):
            return jnp.maximum(m, sv[pl.ds(i * 16, 16)])

        m = lax.fori_loop(0, _K // 16, _gmax,
                          jnp.full((16,), -jnp.inf, jnp.float32))
        mx = jnp.max(m)

        def _gexp(i, acc):
            e = jnp.exp(sv[pl.ds(i * 16, 16)] - mx)
            sv[pl.ds(i * 16, 16)] = e
            return acc + e

        acc = lax.fori_loop(0, _K // 16, _gexp, jnp.zeros((16,), jnp.float32))
        tot = jnp.full((16,), 1.0, jnp.float32) * jnp.sum(acc)

        def _gdiv(i, carry):
            sv[pl.ds(i * 16, 16)] = sv[pl.ds(i * 16, 16)] / tot
            return carry

        lax.fori_loop(0, _K // 16, _gdiv, 0)
        pltpu.sync_copy(sv, w_hbm.at[row])

        _radix_pass(sk, si, rk, ri, hist, 0, 256, _K // 16)
        _radix_pass(rk, ri, sk, si, hist, 8, 256, _K // 16)
        _radix_pass(sk, si, rk, ri, hist, 16, 256, _K // 16)
        _radix_pass(rk, ri, sk, si, hist, 24, 256, _K // 16)
        pltpu.sync_copy(si, topk_hbm.at[row])


def _routing(router_logits):
    nb = router_logits.shape[0]
    mesh = plsc.VectorSubcoreMesh(core_axis_name="c", subcore_axis_name="s")
    fn = pl.kernel(
        _routing_body,
        out_type=[
            jax.ShapeDtypeStruct((nb, _K), jnp.int32),
            jax.ShapeDtypeStruct((nb, _K), jnp.int32),
            jax.ShapeDtypeStruct((nb, _K), jnp.float32),
        ],
        mesh=mesh,
        compiler_params=pltpu.CompilerParams(needs_layout_passes=False),
        scratch_types=[
            pltpu.VMEM((_S,), jnp.float32),
            pltpu.VMEM((_S,), jnp.int32),
            pltpu.VMEM((_K,), jnp.int32),
            pltpu.VMEM((_K,), jnp.int32),
            pltpu.VMEM((_K,), jnp.float32),
            pltpu.VMEM((_K,), jnp.int32),
            pltpu.VMEM((_K,), jnp.int32),
            pltpu.VMEM((256,), jnp.int32),
        ],
    )
    return fn(router_logits)


def kernel(hidden_states, W_router):
    lg0 = _router_logits(hidden_states[:1], W_router)
    tk0, si0, rw0 = _routing(lg0)
    lg1 = _router_logits(hidden_states[1:], W_router)
    tk1, si1, rw1 = _routing(lg1)
    router_logits = jnp.concatenate([lg0, lg1], axis=0)
    sorted_indices = jnp.concatenate([si0, si1], axis=0)
    router_weights = jnp.concatenate([rw0, rw1], axis=0)
    topk_indices = jnp.concatenate([tk0, tk1], axis=0)
    return (sorted_indices, router_weights, router_logits, topk_indices)


# speedup vs baseline: 2.0774x; 2.0774x over previous
import functools

import jax
import jax.numpy as jnp
import numpy as np
from jax import lax
from jax.experimental import pallas as pl
from jax.experimental.pallas import tpu as pltpu
from jax.experimental.pallas import tpu_sc as plsc


_B, _S, _D = 2, 4096, 4096
_BS = 4096
_DK = 1024
_K = 512
_NV = _S // 16


def _matvec_body(x_ref, w_ref, o_ref):
    def _dot(ww, xx):
        return jax.lax.dot_general(
            ww, xx,
            dimension_numbers=(((1,), (1,)), ((), ())),
            preferred_element_type=jnp.float32,
        )

    def _chunk(c):
        lo, hi = 256 * c, 256 * c + 128
        return (_dot(w_ref[:, lo:lo + 128], x_ref[0, :, lo:lo + 128])
                + _dot(w_ref[:, hi:hi + 128], x_ref[0, :, hi:hi + 128]))

    k = pl.program_id(2)

    @pl.when(k == 0)
    def _init():
        a = _chunk(0)
        for c in range(1, _DK // 256):
            a = a + _chunk(c)
        o_ref[0] = a

    @pl.when(k != 0)
    def _acc():
        a = o_ref[0]
        for c in range(_DK // 256):
            a = a + _chunk(c)
        o_ref[0] = a


def _router_logits(hidden_states, W_router):
    wt = W_router.T
    nb = hidden_states.shape[0]
    out = pl.pallas_call(
        _matvec_body,
        grid=(nb, _S // _BS, _D // _DK),
        in_specs=[
            pl.BlockSpec((1, _BS, _DK), lambda b, s, k: (b, s, k)),
            pl.BlockSpec((1, _DK), lambda b, s, k: (0, k)),
        ],
        out_specs=pl.BlockSpec((1, 1, _BS), lambda b, s, k: (b, 0, s)),
        out_shape=jax.ShapeDtypeStruct((nb, 1, _S), jnp.float32),
    )(hidden_states, wt)
    return out[:, 0, :]


def _radix_pass(src_k, src_i, dst_k, dst_i, hist, shift, nbuckets, nveg):
    mask = jnp.int32(nbuckets - 1)
    ones = jnp.ones((16,), jnp.int32)

    for j in range(nbuckets // 16):
        hist[pl.ds(j * 16, 16)] = jnp.zeros((16,), jnp.int32)

    def _hist(i, carry):
        k = src_k[pl.ds(i * 16, 16)]
        d = lax.bitwise_and(lax.shift_right_logical(k, shift), mask)
        plsc.addupdate_scatter(hist, [d], ones)
        return carry

    lax.fori_loop(0, nveg, _hist, 0)

    def _scan(j, carry):
        h = hist[pl.ds(j * 16, 16)]
        inc = plsc.cumsum(h)
        hist[pl.ds(j * 16, 16)] = inc - h + carry
        return carry + jnp.sum(h)

    lax.fori_loop(0, nbuckets // 16, _scan, jnp.int32(0))

    def _scatter(i, carry):
        k = src_k[pl.ds(i * 16, 16)]
        v = src_i[pl.ds(i * 16, 16)]
        d = lax.bitwise_and(lax.shift_right_logical(k, shift), mask)
        cnt, _ = plsc.scan_count(d)
        base = plsc.load_gather(hist, [d])
        p = base + cnt - 1
        plsc.store_scatter(dst_k, [p], k)
        plsc.store_scatter(dst_i, [p], v)
        plsc.addupdate_scatter(hist, [d], ones)
        return carry

    lax.fori_loop(0, nveg, _scatter, 0)


_MIN32 = np.int32(-(2 ** 31))


def _routing_body(logits_hbm, topk_hbm, sorted_hbm, w_hbm,
                  vals, ka, sk, si, sv, rk, ri, hist):
    c = lax.axis_index("c")
    s = lax.axis_index("s")
    nb = logits_hbm.shape[0]

    @pl.when((s == 0) & (c < nb))
    def _route():
        row = c
        pltpu.sync_copy(logits_hbm.at[row], vals)
        ones = jnp.ones((16,), jnp.int32)

        for j in range(16):
            hist[pl.ds(j * 16, 16)] = jnp.zeros((16,), jnp.int32)

        def _keys(i, carry):
            bits = plsc.bitcast(vals[pl.ds(i * 16, 16)], jnp.int32)
            sgn = lax.shift_right_arithmetic(bits, 31)
            k = lax.bitwise_not(
                lax.bitwise_xor(bits, lax.bitwise_or(sgn, _MIN32)))
            ka[pl.ds(i * 16, 16)] = k
            d = lax.shift_right_logical(k, 24)
            plsc.addupdate_scatter(hist, [d], ones)
            return carry

        lax.fori_loop(0, _NV, _keys, 0)

        def _pick(need):
            def _scan(j, carry):
                cum, fb, fe = carry
                h = hist[pl.ds(j * 16, 16)]
                inc = plsc.cumsum(h)
                excl = inc - h + cum
                hit = (excl < need) & (excl + h >= need)
                lane = lax.iota(jnp.int32, 16) + j * 16
                fb = jnp.maximum(fb, jnp.max(jnp.where(hit, lane, -1)))
                fe = jnp.maximum(fe, jnp.max(jnp.where(hit, excl, -1)))
                return cum + jnp.sum(h), fb, fe

            _, fb, fe = lax.fori_loop(
                0, 16, _scan, (jnp.int32(0), jnp.int32(-1), jnp.int32(-1)))
            return fb, fe

        need = jnp.int32(_K)
        b0, fe = _pick(need)
        need = need - fe
        pref = lax.shift_left(b0, 24)

        b0v = jnp.full((16,), 1, jnp.int32) * b0

        def _cand(i, carry):
            base = carry
            k = ka[pl.ds(i * 16, 16)]
            m = lax.shift_right_logical(k, 24) == b0v
            p = base + plsc.cumsum(m.astype(jnp.int32)) - 1
            idx = lax.iota(jnp.int32, 16) + i * 16
            plsc.store_scatter(rk, [p], k, mask=m)
            plsc.store_scatter(ri, [p], idx, mask=m)
            return base + plsc.all_reduce_population_count(m)

        n1v = lax.fori_loop(0, _NV, _cand, jnp.zeros((16,), jnp.int32))
        n1 = jnp.max(n1v)
        nv1 = lax.shift_right_logical(n1 + 15, 4)

        def _level(shift, maskhi, pref, need):
            for j in range(16):
                hist[pl.ds(j * 16, 16)] = jnp.zeros((16,), jnp.int32)

            n1s = jnp.full((16,), 1, jnp.int32) * n1

            def _h(i, carry):
                k = rk[pl.ds(i * 16, 16)]
                lane = lax.iota(jnp.int32, 16) + i * 16
                match = (lax.bitwise_and(k, maskhi) == pref) & (lane < n1s)
                d = lax.bitwise_and(lax.shift_right_logical(k, shift),
                                    jnp.int32(255))
                plsc.addupdate_scatter(hist, [d], ones, mask=match)
                return carry

            lax.fori_loop(0, nv1, _h, 0)
            fb, fe = _pick(need)
            return pref | lax.shift_left(fb, shift), need - fe

        pref, need = _level(16, jnp.int32(-(2 ** 24)), pref, need)
        pref, need = _level(8, jnp.int32(-(2 ** 16)), pref, need)
        pref, need = _level(0, jnp.int32(-(2 ** 8)), pref, need)

        tf = lax.bitwise_xor(pref, _MIN32)
        needv = jnp.full((16,), 1, jnp.int32) * need

        def _compact(i, carry):
            base, eqb = carry
            k = ka[pl.ds(i * 16, 16)]
            v = vals[pl.ds(i * 16, 16)]
            idx = lax.iota(jnp.int32, 16) + i * 16
            m_lt = lax.bitwise_xor(k, _MIN32) < tf
            m_eq = k == pref
            ceq = plsc.cumsum(m_eq.astype(jnp.int32))
            m = m_lt | (m_eq & ((eqb + ceq) <= needv))
            p = base + plsc.cumsum(m.astype(jnp.int32)) - 1
            plsc.store_scatter(sk, [p], k, mask=m)
            plsc.store_scatter(si, [p], idx, mask=m)
            plsc.store_scatter(sv, [p], v, mask=m)
            return (base + plsc.all_reduce_population_count(m),
                    eqb + plsc.all_reduce_population_count(m_eq))

        lax.fori_loop(0, _NV, _compact,
                      (jnp.zeros((16,), jnp.int32), jnp.zeros((16,), jnp.int32)))

        pltpu.sync_copy(si, sorted_hbm.at[row])

        def _gmax(i, m):
            return jnp.maximum(m, sv[pl.ds(i * 16, 16)])

        m = lax.fori_loop(0, _K // 16, _gmax,
                          jnp.full((16,), -jnp.inf, jnp.float32))
        mx = jnp.max(m)

        def _gexp(i, acc):
            e = jnp.exp(sv[pl.ds(i * 16, 16)] - mx)
            sv[pl.ds(i * 16, 16)] = e
            return acc + e

        acc = lax.fori_loop(0, _K // 16, _gexp, jnp.zeros((16,), jnp.float32))
        tot = jnp.full((16,), 1.0, jnp.float32) * jnp.sum(acc)

        def _gdiv(i, carry):
            sv[pl.ds(i * 16, 16)] = sv[pl.ds(i * 16, 16)] / tot
            return carry

        lax.fori_loop(0, _K // 16, _gdiv, 0)
        pltpu.sync_copy(sv, w_hbm.at[row])

        _radix_pass(sk, si, rk, ri, hist, 0, 256, _K // 16)
        _radix_pass(rk, ri, sk, si, hist, 8, 256, _K // 16)
        _radix_pass(sk, si, rk, ri, hist, 16, 256, _K // 16)
        _radix_pass(rk, ri, sk, si, hist, 24, 256, _K // 16)
        pltpu.sync_copy(si, topk_hbm.at[row])


def _routing(router_logits):
    nb = router_logits.shape[0]
    mesh = plsc.VectorSubcoreMesh(core_axis_name="c", subcore_axis_name="s")
    fn = pl.kernel(
        _routing_body,
        out_type=[
            jax.ShapeDtypeStruct((nb, _K), jnp.int32),
            jax.ShapeDtypeStruct((nb, _K), jnp.int32),
            jax.ShapeDtypeStruct((nb, _K), jnp.float32),
        ],
        mesh=mesh,
        compiler_params=pltpu.CompilerParams(needs_layout_passes=False),
        scratch_types=[
            pltpu.VMEM((_S,), jnp.float32),
            pltpu.VMEM((_S,), jnp.int32),
            pltpu.VMEM((_K,), jnp.int32),
            pltpu.VMEM((_K,), jnp.int32),
            pltpu.VMEM((_K,), jnp.float32),
            pltpu.VMEM((_S,), jnp.int32),
            pltpu.VMEM((_S,), jnp.int32),
            pltpu.VMEM((256,), jnp.int32),
        ],
    )
    return fn(router_logits)


def kernel(hidden_states, W_router):
    router_logits = _router_logits(hidden_states, W_router)
    topk_indices, sorted_indices, router_weights = _routing(router_logits)
    return (sorted_indices, router_weights, router_logits, topk_indices)


# speedup vs baseline: 2.1064x; 1.0139x over previous
import functools

import jax
import jax.numpy as jnp
import numpy as np
from jax import lax
from jax.experimental import pallas as pl
from jax.experimental.pallas import tpu as pltpu
from jax.experimental.pallas import tpu_sc as plsc


_B, _S, _D = 2, 4096, 4096
_DK = 512
_K = 512
_NV = _S // 16


def _matvec_body(x_ref, w_ref, o_ref):
    def _dot(ww, xx):
        return jax.lax.dot_general(
            ww, xx,
            dimension_numbers=(((1,), (1,)), ((), ())),
            preferred_element_type=jnp.float32,
        )

    def _chunk(c):
        lo, hi = 256 * c, 256 * c + 128
        return (_dot(w_ref[:, lo:lo + 128], x_ref[:, lo:lo + 128])
                + _dot(w_ref[:, hi:hi + 128], x_ref[:, hi:hi + 128]))

    k = pl.program_id(0)

    @pl.when(k == 0)
    def _init():
        a = _chunk(0)
        for c in range(1, _DK // 256):
            a = a + _chunk(c)
        o_ref[...] = a

    @pl.when(k != 0)
    def _acc():
        a = o_ref[...]
        for c in range(_DK // 256):
            a = a + _chunk(c)
        o_ref[...] = a


def _router_logits(hidden_states, W_router):
    wt = W_router.T
    nb = hidden_states.shape[0]
    xf = hidden_states.reshape(nb * _S, _D)
    out = pl.pallas_call(
        _matvec_body,
        grid=(_D // _DK,),
        in_specs=[
            pl.BlockSpec((nb * _S, _DK), lambda k: (0, k)),
            pl.BlockSpec((1, _DK), lambda k: (0, k)),
        ],
        out_specs=pl.BlockSpec((1, nb * _S), lambda k: (0, 0)),
        out_shape=jax.ShapeDtypeStruct((1, nb * _S), jnp.float32),
    )(xf, wt)
    return out.reshape(nb, _S)


def _radix_pass(src_k, src_i, dst_k, dst_i, hist, shift, nbuckets, nveg):
    mask = jnp.int32(nbuckets - 1)
    ones = jnp.ones((16,), jnp.int32)

    for j in range(nbuckets // 16):
        hist[pl.ds(j * 16, 16)] = jnp.zeros((16,), jnp.int32)

    def _hist(i, carry):
        k = src_k[pl.ds(i * 16, 16)]
        d = lax.bitwise_and(lax.shift_right_logical(k, shift), mask)
        plsc.addupdate_scatter(hist, [d], ones)
        return carry

    lax.fori_loop(0, nveg, _hist, 0)

    def _scan(j, carry):
        h = hist[pl.ds(j * 16, 16)]
        inc = plsc.cumsum(h)
        hist[pl.ds(j * 16, 16)] = inc - h + carry
        return carry + jnp.sum(h)

    lax.fori_loop(0, nbuckets // 16, _scan, jnp.int32(0))

    def _scatter(i, carry):
        k = src_k[pl.ds(i * 16, 16)]
        v = src_i[pl.ds(i * 16, 16)]
        d = lax.bitwise_and(lax.shift_right_logical(k, shift), mask)
        cnt, _ = plsc.scan_count(d)
        base = plsc.load_gather(hist, [d])
        p = base + cnt - 1
        plsc.store_scatter(dst_k, [p], k)
        plsc.store_scatter(dst_i, [p], v)
        plsc.addupdate_scatter(hist, [d], ones)
        return carry

    lax.fori_loop(0, nveg, _scatter, 0)


_MIN32 = np.int32(-(2 ** 31))


def _routing_body(logits_hbm, topk_hbm, sorted_hbm, w_hbm,
                  vals, ka, sk, si, sv, rk, ri, hist):
    c = lax.axis_index("c")
    s = lax.axis_index("s")
    nb = logits_hbm.shape[0]

    @pl.when((s == 0) & (c < nb))
    def _route():
        row = c
        pltpu.sync_copy(logits_hbm.at[row], vals)
        ones = jnp.ones((16,), jnp.int32)

        for j in range(16):
            hist[pl.ds(j * 16, 16)] = jnp.zeros((16,), jnp.int32)

        def _keys(i, carry):
            bits = plsc.bitcast(vals[pl.ds(i * 16, 16)], jnp.int32)
            sgn = lax.shift_right_arithmetic(bits, 31)
            k = lax.bitwise_not(
                lax.bitwise_xor(bits, lax.bitwise_or(sgn, _MIN32)))
            ka[pl.ds(i * 16, 16)] = k
            d = lax.shift_right_logical(k, 24)
            plsc.addupdate_scatter(hist, [d], ones)
            return carry

        lax.fori_loop(0, _NV, _keys, 0)

        def _pick(need):
            def _scan(j, carry):
                cum, fb, fe = carry
                h = hist[pl.ds(j * 16, 16)]
                inc = plsc.cumsum(h)
                excl = inc - h + cum
                hit = (excl < need) & (excl + h >= need)
                lane = lax.iota(jnp.int32, 16) + j * 16
                fb = jnp.maximum(fb, jnp.max(jnp.where(hit, lane, -1)))
                fe = jnp.maximum(fe, jnp.max(jnp.where(hit, excl, -1)))
                return cum + jnp.sum(h), fb, fe

            _, fb, fe = lax.fori_loop(
                0, 16, _scan, (jnp.int32(0), jnp.int32(-1), jnp.int32(-1)))
            return fb, fe

        need = jnp.int32(_K)
        b0, fe = _pick(need)
        need = need - fe
        pref = lax.shift_left(b0, 24)

        b0v = jnp.full((16,), 1, jnp.int32) * b0

        def _cand(i, carry):
            base = carry
            k = ka[pl.ds(i * 16, 16)]
            m = lax.shift_right_logical(k, 24) == b0v
            p = base + plsc.cumsum(m.astype(jnp.int32)) - 1
            idx = lax.iota(jnp.int32, 16) + i * 16
            plsc.store_scatter(rk, [p], k, mask=m)
            plsc.store_scatter(ri, [p], idx, mask=m)
            return base + plsc.all_reduce_population_count(m)

        n1v = lax.fori_loop(0, _NV, _cand, jnp.zeros((16,), jnp.int32))
        n1 = jnp.max(n1v)
        nv1 = lax.shift_right_logical(n1 + 15, 4)

        def _level(shift, maskhi, pref, need):
            for j in range(16):
                hist[pl.ds(j * 16, 16)] = jnp.zeros((16,), jnp.int32)

            n1s = jnp.full((16,), 1, jnp.int32) * n1

            def _h(i, carry):
                k = rk[pl.ds(i * 16, 16)]
                lane = lax.iota(jnp.int32, 16) + i * 16
                match = (lax.bitwise_and(k, maskhi) == pref) & (lane < n1s)
                d = lax.bitwise_and(lax.shift_right_logical(k, shift),
                                    jnp.int32(255))
                plsc.addupdate_scatter(hist, [d], ones, mask=match)
                return carry

            lax.fori_loop(0, nv1, _h, 0)
            fb, fe = _pick(need)
            return pref | lax.shift_left(fb, shift), need - fe

        pref, need = _level(16, jnp.int32(-(2 ** 24)), pref, need)
        pref, need = _level(8, jnp.int32(-(2 ** 16)), pref, need)
        pref, need = _level(0, jnp.int32(-(2 ** 8)), pref, need)

        tf = lax.bitwise_xor(pref, _MIN32)
        needv = jnp.full((16,), 1, jnp.int32) * need

        def _compact(i, carry):
            base, eqb = carry
            k = ka[pl.ds(i * 16, 16)]
            v = vals[pl.ds(i * 16, 16)]
            idx = lax.iota(jnp.int32, 16) + i * 16
            m_lt = lax.bitwise_xor(k, _MIN32) < tf
            m_eq = k == pref
            ceq = plsc.cumsum(m_eq.astype(jnp.int32))
            m = m_lt | (m_eq & ((eqb + ceq) <= needv))
            p = base + plsc.cumsum(m.astype(jnp.int32)) - 1
            plsc.store_scatter(sk, [p], k, mask=m)
            plsc.store_scatter(si, [p], idx, mask=m)
            plsc.store_scatter(sv, [p], v, mask=m)
            return (base + plsc.all_reduce_population_count(m),
                    eqb + plsc.all_reduce_population_count(m_eq))

        lax.fori_loop(0, _NV, _compact,
                      (jnp.zeros((16,), jnp.int32), jnp.zeros((16,), jnp.int32)))

        pltpu.sync_copy(si, sorted_hbm.at[row])

        def _gmax(i, m):
            return jnp.maximum(m, sv[pl.ds(i * 16, 16)])

        m = lax.fori_loop(0, _K // 16, _gmax,
                          jnp.full((16,), -jnp.inf, jnp.float32))
        mx = jnp.max(m)

        def _gexp(i, acc):
            e = jnp.exp(sv[pl.ds(i * 16, 16)] - mx)
            sv[pl.ds(i * 16, 16)] = e
            return acc + e

        acc = lax.fori_loop(0, _K // 16, _gexp, jnp.zeros((16,), jnp.float32))
        tot = jnp.full((16,), 1.0, jnp.float32) * jnp.sum(acc)

        def _gdiv(i, carry):
            sv[pl.ds(i * 16, 16)] = sv[pl.ds(i * 16, 16)] / tot
            return carry

        lax.fori_loop(0, _K // 16, _gdiv, 0)
        pltpu.sync_copy(sv, w_hbm.at[row])

        _radix_pass(sk, si, rk, ri, hist, 0, 256, _K // 16)
        _radix_pass(rk, ri, sk, si, hist, 8, 256, _K // 16)
        _radix_pass(sk, si, rk, ri, hist, 16, 256, _K // 16)
        _radix_pass(rk, ri, sk, si, hist, 24, 256, _K // 16)
        pltpu.sync_copy(si, topk_hbm.at[row])


def _routing(router_logits):
    nb = router_logits.shape[0]
    mesh = plsc.VectorSubcoreMesh(core_axis_name="c", subcore_axis_name="s")
    fn = pl.kernel(
        _routing_body,
        out_type=[
            jax.ShapeDtypeStruct((nb, _K), jnp.int32),
            jax.ShapeDtypeStruct((nb, _K), jnp.int32),
            jax.ShapeDtypeStruct((nb, _K), jnp.float32),
        ],
        mesh=mesh,
        compiler_params=pltpu.CompilerParams(needs_layout_passes=False),
        scratch_types=[
            pltpu.VMEM((_S,), jnp.float32),
            pltpu.VMEM((_S,), jnp.int32),
            pltpu.VMEM((_K,), jnp.int32),
            pltpu.VMEM((_K,), jnp.int32),
            pltpu.VMEM((_K,), jnp.float32),
            pltpu.VMEM((_S,), jnp.int32),
            pltpu.VMEM((_S,), jnp.int32),
            pltpu.VMEM((256,), jnp.int32),
        ],
    )
    return fn(router_logits)


def kernel(hidden_states, W_router):
    router_logits = _router_logits(hidden_states, W_router)
    topk_indices, sorted_indices, router_weights = _routing(router_logits)
    return (sorted_indices, router_weights, router_logits, topk_indices)


# speedup vs baseline: 2.1140x; 1.0036x over previous
import functools

import jax
import jax.numpy as jnp
import numpy as np
from jax import lax
from jax.experimental import pallas as pl
from jax.experimental.pallas import tpu as pltpu
from jax.experimental.pallas import tpu_sc as plsc


_B, _S, _D = 2, 4096, 4096
_DK = 512
_K = 512
_NV = _S // 16


def _matvec_body(x_ref, w_ref, o_ref):
    def _dot(ww, xx):
        return jax.lax.dot_general(
            ww, xx,
            dimension_numbers=(((1,), (1,)), ((), ())),
            preferred_element_type=jnp.float32,
        )

    def _chunk(c):
        lo, hi = 256 * c, 256 * c + 128
        return (_dot(w_ref[:, lo:lo + 128], x_ref[:, lo:lo + 128])
                + _dot(w_ref[:, hi:hi + 128], x_ref[:, hi:hi + 128]))

    k = pl.program_id(0)

    @pl.when(k == 0)
    def _init():
        a = _chunk(0)
        for c in range(1, _DK // 256):
            a = a + _chunk(c)
        o_ref[...] = a

    @pl.when(k != 0)
    def _acc():
        a = o_ref[...]
        for c in range(_DK // 256):
            a = a + _chunk(c)
        o_ref[...] = a


def _router_logits(hidden_states, W_router):
    wt = W_router.T
    nb = hidden_states.shape[0]
    xf = hidden_states.reshape(nb * _S, _D)
    out = pl.pallas_call(
        _matvec_body,
        grid=(_D // _DK,),
        in_specs=[
            pl.BlockSpec((nb * _S, _DK), lambda k: (0, k)),
            pl.BlockSpec((1, _DK), lambda k: (0, k)),
        ],
        out_specs=pl.BlockSpec((1, nb * _S), lambda k: (0, 0)),
        out_shape=jax.ShapeDtypeStruct((1, nb * _S), jnp.float32),
        compiler_params=pltpu.CompilerParams(
            fuse_transposed_lhs_in_matmul=True),
    )(xf, wt)
    return out.reshape(nb, _S)


def _radix_pass(src_k, src_i, dst_k, dst_i, hist, shift, nbuckets, nveg):
    mask = jnp.int32(nbuckets - 1)
    ones = jnp.ones((16,), jnp.int32)

    for j in range(nbuckets // 16):
        hist[pl.ds(j * 16, 16)] = jnp.zeros((16,), jnp.int32)

    def _hist(i, carry):
        k = src_k[pl.ds(i * 16, 16)]
        d = lax.bitwise_and(lax.shift_right_logical(k, shift), mask)
        plsc.addupdate_scatter(hist, [d], ones)
        return carry

    lax.fori_loop(0, nveg, _hist, 0)

    def _scan(j, carry):
        h = hist[pl.ds(j * 16, 16)]
        inc = plsc.cumsum(h)
        hist[pl.ds(j * 16, 16)] = inc - h + carry
        return carry + jnp.sum(h)

    lax.fori_loop(0, nbuckets // 16, _scan, jnp.int32(0))

    def _scatter(i, carry):
        k = src_k[pl.ds(i * 16, 16)]
        v = src_i[pl.ds(i * 16, 16)]
        d = lax.bitwise_and(lax.shift_right_logical(k, shift), mask)
        cnt, _ = plsc.scan_count(d)
        base = plsc.load_gather(hist, [d])
        p = base + cnt - 1
        plsc.store_scatter(dst_k, [p], k)
        plsc.store_scatter(dst_i, [p], v)
        plsc.addupdate_scatter(hist, [d], ones)
        return carry

    lax.fori_loop(0, nveg, _scatter, 0)


_MIN32 = np.int32(-(2 ** 31))


def _routing_body(logits_hbm, topk_hbm, sorted_hbm, w_hbm,
                  vals, ka, sk, si, sv, rk, ri, hist):
    c = lax.axis_index("c")
    s = lax.axis_index("s")
    nb = logits_hbm.shape[0]

    @pl.when((s == 0) & (c < nb))
    def _route():
        row = c
        pltpu.sync_copy(logits_hbm.at[row], vals)
        ones = jnp.ones((16,), jnp.int32)

        for j in range(16):
            hist[pl.ds(j * 16, 16)] = jnp.zeros((16,), jnp.int32)

        def _keys(i, carry):
            bits = plsc.bitcast(vals[pl.ds(i * 16, 16)], jnp.int32)
            sgn = lax.shift_right_arithmetic(bits, 31)
            k = lax.bitwise_not(
                lax.bitwise_xor(bits, lax.bitwise_or(sgn, _MIN32)))
            ka[pl.ds(i * 16, 16)] = k
            d = lax.shift_right_logical(k, 24)
            plsc.addupdate_scatter(hist, [d], ones)
            return carry

        lax.fori_loop(0, _NV, _keys, 0)

        def _pick(need):
            def _scan(j, carry):
                cum, fb, fe = carry
                h = hist[pl.ds(j * 16, 16)]
                inc = plsc.cumsum(h)
                excl = inc - h + cum
                hit = (excl < need) & (excl + h >= need)
                lane = lax.iota(jnp.int32, 16) + j * 16
                fb = jnp.maximum(fb, jnp.max(jnp.where(hit, lane, -1)))
                fe = jnp.maximum(fe, jnp.max(jnp.where(hit, excl, -1)))
                return cum + jnp.sum(h), fb, fe

            _, fb, fe = lax.fori_loop(
                0, 16, _scan, (jnp.int32(0), jnp.int32(-1), jnp.int32(-1)))
            return fb, fe

        need = jnp.int32(_K)
        b0, fe = _pick(need)
        need = need - fe
        pref = lax.shift_left(b0, 24)

        b0v = jnp.full((16,), 1, jnp.int32) * b0

        def _cand(i, carry):
            base = carry
            k = ka[pl.ds(i * 16, 16)]
            m = lax.shift_right_logical(k, 24) == b0v
            p = base + plsc.cumsum(m.astype(jnp.int32)) - 1
            idx = lax.iota(jnp.int32, 16) + i * 16
            plsc.store_scatter(rk, [p], k, mask=m)
            plsc.store_scatter(ri, [p], idx, mask=m)
            return base + plsc.all_reduce_population_count(m)

        n1v = lax.fori_loop(0, _NV, _cand, jnp.zeros((16,), jnp.int32))
        n1 = jnp.max(n1v)
        nv1 = lax.shift_right_logical(n1 + 15, 4)

        def _level(shift, maskhi, pref, need):
            for j in range(16):
                hist[pl.ds(j * 16, 16)] = jnp.zeros((16,), jnp.int32)

            n1s = jnp.full((16,), 1, jnp.int32) * n1

            def _h(i, carry):
                k = rk[pl.ds(i * 16, 16)]
                lane = lax.iota(jnp.int32, 16) + i * 16
                match = (lax.bitwise_and(k, maskhi) == pref) & (lane < n1s)
                d = lax.bitwise_and(lax.shift_right_logical(k, shift),
                                    jnp.int32(255))
                plsc.addupdate_scatter(hist, [d], ones, mask=match)
                return carry

            lax.fori_loop(0, nv1, _h, 0)
            fb, fe = _pick(need)
            return pref | lax.shift_left(fb, shift), need - fe

        pref, need = _level(16, jnp.int32(-(2 ** 24)), pref, need)
        pref, need = _level(8, jnp.int32(-(2 ** 16)), pref, need)
        pref, need = _level(0, jnp.int32(-(2 ** 8)), pref, need)

        tf = lax.bitwise_xor(pref, _MIN32)
        needv = jnp.full((16,), 1, jnp.int32) * need

        def _compact(i, carry):
            base, eqb = carry
            k = ka[pl.ds(i * 16, 16)]
            v = vals[pl.ds(i * 16, 16)]
            idx = lax.iota(jnp.int32, 16) + i * 16
            m_lt = lax.bitwise_xor(k, _MIN32) < tf
            m_eq = k == pref
            ceq = plsc.cumsum(m_eq.astype(jnp.int32))
            m = m_lt | (m_eq & ((eqb + ceq) <= needv))
            p = base + plsc.cumsum(m.astype(jnp.int32)) - 1
            plsc.store_scatter(sk, [p], k, mask=m)
            plsc.store_scatter(si, [p], idx, mask=m)
            plsc.store_scatter(sv, [p], v, mask=m)
            return (base + plsc.all_reduce_population_count(m),
                    eqb + plsc.all_reduce_population_count(m_eq))

        lax.fori_loop(0, _NV, _compact,
                      (jnp.zeros((16,), jnp.int32), jnp.zeros((16,), jnp.int32)))

        pltpu.sync_copy(si, sorted_hbm.at[row])

        def _gmax(i, m):
            return jnp.maximum(m, sv[pl.ds(i * 16, 16)])

        m = lax.fori_loop(0, _K // 16, _gmax,
                          jnp.full((16,), -jnp.inf, jnp.float32))
        mx = jnp.max(m)

        def _gexp(i, acc):
            e = jnp.exp(sv[pl.ds(i * 16, 16)] - mx)
            sv[pl.ds(i * 16, 16)] = e
            return acc + e

        acc = lax.fori_loop(0, _K // 16, _gexp, jnp.zeros((16,), jnp.float32))
        tot = jnp.full((16,), 1.0, jnp.float32) * jnp.sum(acc)

        def _gdiv(i, carry):
            sv[pl.ds(i * 16, 16)] = sv[pl.ds(i * 16, 16)] / tot
            return carry

        lax.fori_loop(0, _K // 16, _gdiv, 0)
        pltpu.sync_copy(sv, w_hbm.at[row])

        _radix_pass(sk, si, rk, ri, hist, 0, 256, _K // 16)
        _radix_pass(rk, ri, sk, si, hist, 8, 256, _K // 16)
        _radix_pass(sk, si, rk, ri, hist, 16, 256, _K // 16)
        _radix_pass(rk, ri, sk, si, hist, 24, 256, _K // 16)
        pltpu.sync_copy(si, topk_hbm.at[row])


def _routing(router_logits):
    nb = router_logits.shape[0]
    mesh = plsc.VectorSubcoreMesh(core_axis_name="c", subcore_axis_name="s")
    fn = pl.kernel(
        _routing_body,
        out_type=[
            jax.ShapeDtypeStruct((nb, _K), jnp.int32),
            jax.ShapeDtypeStruct((nb, _K), jnp.int32),
            jax.ShapeDtypeStruct((nb, _K), jnp.float32),
        ],
        mesh=mesh,
        compiler_params=pltpu.CompilerParams(needs_layout_passes=False),
        scratch_types=[
            pltpu.VMEM((_S,), jnp.float32),
            pltpu.VMEM((_S,), jnp.int32),
            pltpu.VMEM((_K,), jnp.int32),
            pltpu.VMEM((_K,), jnp.int32),
            pltpu.VMEM((_K,), jnp.float32),
            pltpu.VMEM((_S,), jnp.int32),
            pltpu.VMEM((_S,), jnp.int32),
            pltpu.VMEM((256,), jnp.int32),
        ],
    )
    return fn(router_logits)


def kernel(hidden_states, W_router):
    router_logits = _router_logits(hidden_states, W_router)
    topk_indices, sorted_indices, router_weights = _routing(router_logits)
    return (sorted_indices, router_weights, router_logits, topk_indices)
